# baseline (speedup 1.0000x reference)
import functools

import jax
import jax.numpy as jnp
from jax import lax
from jax.experimental import pallas as pl
from jax.experimental.pallas import tpu as pltpu
from jax.experimental.pallas import tpu_sc as plsc

N_EMB = 1000000
D = 64
B0, B1 = 4096, 200
N_ROWS = B0 * B1

_info = plsc.get_sparse_core_info()
NC, NS = _info.num_cores, _info.num_subcores
NW = NC * NS
B_PER_W = N_ROWS // NW
CHUNK = 512
G = 1
GROUP = G * CHUNK
SETS = 3
NG = B_PER_W // GROUP
N_CHUNKS = B_PER_W // CHUNK

_mesh = plsc.VectorSubcoreMesh(core_axis_name="c", subcore_axis_name="s")

_CB = 16384
_NFULL = N_EMB // _CB
_TAIL0 = _NFULL * _CB
_TAILN = N_EMB - _TAIL0
_TH = _TAILN // 2


def _pack_body(in_ref, out_ref):
    c = pl.program_id(0)
    x = in_ref[...]

    @pl.when(c < _NFULL)
    def _():
        out_ref[:, 0:D] = x[:, 0 : _CB // 2].T
        out_ref[:, D : 2 * D] = x[:, _CB // 2 : _CB].T

    @pl.when(c == _NFULL)
    def _():
        out_ref[0:_TH, 0:D] = x[:, 0:_TH].T
        out_ref[0:_TH, D : 2 * D] = x[:, _TH:_TAILN].T


def _tc_pack(table_t):
    return pl.pallas_call(
        _pack_body,
        grid=(_NFULL + 1,),
        in_specs=[pl.BlockSpec((D, _CB), lambda c: (0, c))],
        out_specs=pl.BlockSpec((_CB // 2, 2 * D), lambda c: (c, 0)),
        out_shape=jax.ShapeDtypeStruct((N_EMB // 2, 2 * D), jnp.float32),
    )(table_t)


def _pi(t):
    c, tau = t // _CB, t % _CB
    full = c * _CB + 2 * (tau % (_CB // 2)) + tau // (_CB // 2)
    tau2 = t - _TAIL0
    tail = _TAIL0 + 2 * (tau2 % _TH) + tau2 // _TH
    return jnp.where(t < _TAIL0, full, tail)


NGR = 50
OCT = 256
NOCT = B1 * 8
NSPLIT = 2
NGR_H = NGR // NSPLIT


def _make_gather(ngr):
    @functools.partial(
        pl.kernel,
        mesh=_mesh,
        out_type=jax.ShapeDtypeStruct((ngr * NW * OCT, 2 * D), jnp.float32),
        compiler_params=pltpu.CompilerParams(use_tc_tiling_on_sc=False),
        scratch_types=[
            pltpu.VMEM((ngr, OCT), jnp.int32),
            pltpu.VMEM((ngr, OCT), jnp.int32),
            pltpu.VMEM((SETS * 2 * OCT, D), jnp.float32),
            pltpu.SemaphoreType.DMA((SETS,)),
            pltpu.SemaphoreType.DMA((SETS,)),
        ],
    )
    def gather(idxa_hbm, idxb_hbm, table_hbm, out_hbm, iva, ivb, rows_v,
               gsem, osem):
        wid = lax.axis_index("s") * NC + lax.axis_index("c")
        gbase = wid * ngr
        pltpu.sync_copy(idxa_hbm.at[pl.ds(gbase, ngr)], iva)
        pltpu.sync_copy(idxb_hbm.at[pl.ds(gbase, ngr)], ivb)

        def fire_gathers(g, s):
            pltpu.async_copy(
                table_hbm.at[iva.at[g]],
                rows_v.at[pl.ds(s * 2 * OCT, OCT)],
                gsem.at[s],
            )
            pltpu.async_copy(
                table_hbm.at[ivb.at[g]],
                rows_v.at[pl.ds(s * 2 * OCT + OCT, OCT)],
                gsem.at[s],
            )

        def drain_gathers(s):
            for q in range(2):
                pltpu.make_async_copy(
                    table_hbm.at[pl.ds(0, OCT)],
                    rows_v.at[pl.ds(s * 2 * OCT + q * OCT, OCT)],
                    gsem.at[s],
                ).wait()

        def fire_wb(g, s):
            pb = (gbase + g) * OCT
            for q in range(2):
                pltpu.async_copy(
                    rows_v.at[pl.ds(s * 2 * OCT + q * OCT, OCT)],
                    out_hbm.at[pl.ds(pb, OCT), pl.ds(q * D, D)],
                    osem.at[s],
                )

        def drain_wb(s):
            for q in range(2):
                pltpu.make_async_copy(
                    out_hbm.at[pl.ds(0, OCT), pl.ds(q * D, D)],
                    rows_v.at[pl.ds(s * 2 * OCT + q * OCT, OCT)],
                    osem.at[s],
                ).wait()

        fire_gathers(0, 0)
        fire_gathers(1, 1)

        def step(g, carry):
            s = g % SETS
            s2 = (g + 2) % SETS
            drain_gathers(s)
            fire_wb(g, s)

            @pl.when(g >= 1)
            def _():
                drain_wb(s2)

            @pl.when(g < ngr - 2)
            def _():
                fire_gathers(g + 2, s2)

            return carry

        lax.fori_loop(0, ngr, step, 0)
        drain_wb((ngr - 1) % SETS)

    return gather


_gather_half = _make_gather(NGR_H)


_BH = B1 // NSPLIT


def _unpack_body(ina_ref, inb_ref, out_ref):
    b = pl.program_id(0)

    @pl.when(b < _BH)
    def _():
        x = ina_ref[0]
        out_ref[0, :, 0 : B0 // 2] = x[:, 0:D].T
        out_ref[0, :, B0 // 2 : B0] = x[:, D : 2 * D].T

    @pl.when(b >= _BH)
    def _():
        x = inb_ref[0]
        out_ref[0, :, 0 : B0 // 2] = x[:, 0:D].T
        out_ref[0, :, B0 // 2 : B0] = x[:, D : 2 * D].T


def _tc_unpack(pa, pb):
    return pl.pallas_call(
        _unpack_body,
        grid=(B1,),
        in_specs=[
            pl.BlockSpec(
                (1, B0 // 2, 2 * D),
                lambda b: (jnp.minimum(b, _BH - 1), 0, 0),
            ),
            pl.BlockSpec(
                (1, B0 // 2, 2 * D),
                lambda b: (jnp.maximum(b - _BH, 0), 0, 0),
            ),
        ],
        out_specs=pl.BlockSpec((1, D, B0), lambda b: (b, 0, 0)),
        out_shape=jax.ShapeDtypeStruct((B1, D, B0), jnp.float32),
    )(pa, pb)


def kernel(x, table):
    xt = x.T
    idxa = _pi(xt[:, : B0 // 2]).reshape(NOCT, OCT)
    idxb = _pi(xt[:, B0 // 2 :]).reshape(NOCT, OCT)

    t_pairs = _tc_pack(table.T)
    tp = t_pairs.reshape(N_EMB, D)
    half = NOCT // NSPLIT
    out_a = _gather_half(idxa[:half], idxb[:half], tp)
    out_b = _gather_half(idxa[half:], idxb[half:], tp)

    pa = out_a.reshape(_BH, B0 // 2, 2 * D)
    pb = out_b.reshape(_BH, B0 // 2, 2 * D)
    res = _tc_unpack(pa, pb)
    return res.transpose(2, 0, 1)


# speedup vs baseline: 1.0614x; 1.0614x over previous
import functools

import jax
import jax.numpy as jnp
from jax import lax
from jax.experimental import pallas as pl
from jax.experimental.pallas import tpu as pltpu
from jax.experimental.pallas import tpu_sc as plsc

N_EMB = 1000000
D = 64
B0, B1 = 4096, 200
N_ROWS = B0 * B1

_info = plsc.get_sparse_core_info()
NC, NS = _info.num_cores, _info.num_subcores
NW = NC * NS
B_PER_W = N_ROWS // NW
CHUNK = 512
G = 1
GROUP = G * CHUNK
SETS = 3
NG = B_PER_W // GROUP
N_CHUNKS = B_PER_W // CHUNK

_mesh = plsc.VectorSubcoreMesh(core_axis_name="c", subcore_axis_name="s")

_CB = 16384
_NFULL = N_EMB // _CB
_TAIL0 = _NFULL * _CB
_TAILN = N_EMB - _TAIL0
_TH = _TAILN // 2


def _pack_body(in_ref, out_ref):
    c = pl.program_id(0)
    x = in_ref[...]

    @pl.when(c < _NFULL)
    def _():
        out_ref[:, 0:D] = x[:, 0 : _CB // 2].T
        out_ref[:, D : 2 * D] = x[:, _CB // 2 : _CB].T

    @pl.when(c == _NFULL)
    def _():
        out_ref[0:_TH, 0:D] = x[:, 0:_TH].T
        out_ref[0:_TH, D : 2 * D] = x[:, _TH:_TAILN].T


def _tc_pack(table_t):
    return pl.pallas_call(
        _pack_body,
        grid=(_NFULL + 1,),
        in_specs=[pl.BlockSpec((D, _CB), lambda c: (0, c))],
        out_specs=pl.BlockSpec((_CB // 2, 2 * D), lambda c: (c, 0)),
        out_shape=jax.ShapeDtypeStruct((N_EMB // 2, 2 * D), jnp.float32),
    )(table_t)


def _pi(t):
    c, tau = t // _CB, t % _CB
    full = c * _CB + 2 * (tau % (_CB // 2)) + tau // (_CB // 2)
    tau2 = t - _TAIL0
    tail = _TAIL0 + 2 * (tau2 % _TH) + tau2 // _TH
    return jnp.where(t < _TAIL0, full, tail)


NGR = 50
OCT = 256
NOCT = B1 * 8
NSPLIT = 2
NGR_H = NGR // NSPLIT


def _make_gather(ngr):
    @functools.partial(
        pl.kernel,
        mesh=_mesh,
        out_type=jax.ShapeDtypeStruct((ngr * NW * OCT, 2 * D), jnp.float32),
        compiler_params=pltpu.CompilerParams(use_tc_tiling_on_sc=False),
        scratch_types=[
            pltpu.VMEM((ngr, OCT), jnp.int32),
            pltpu.VMEM((ngr, OCT), jnp.int32),
            pltpu.VMEM((SETS * 2 * OCT, D), jnp.float32),
            pltpu.SemaphoreType.DMA((SETS,)),
            pltpu.SemaphoreType.DMA((SETS,)),
        ],
    )
    def gather(idxa_hbm, idxb_hbm, table_hbm, out_hbm, iva, ivb, rows_v,
               gsem, osem):
        wid = lax.axis_index("s") * NC + lax.axis_index("c")
        gbase = wid * ngr
        pltpu.sync_copy(idxa_hbm.at[pl.ds(gbase, ngr)], iva)
        pltpu.sync_copy(idxb_hbm.at[pl.ds(gbase, ngr)], ivb)

        def fire_gathers(g, s):
            pltpu.async_copy(
                table_hbm.at[iva.at[g]],
                rows_v.at[pl.ds(s * 2 * OCT, OCT)],
                gsem.at[s],
            )
            pltpu.async_copy(
                table_hbm.at[ivb.at[g]],
                rows_v.at[pl.ds(s * 2 * OCT + OCT, OCT)],
                gsem.at[s],
            )

        def drain_gathers(s):
            for q in range(2):
                pltpu.make_async_copy(
                    table_hbm.at[pl.ds(0, OCT)],
                    rows_v.at[pl.ds(s * 2 * OCT + q * OCT, OCT)],
                    gsem.at[s],
                ).wait()

        def fire_wb(g, s):
            pb = (gbase + g) * OCT
            for q in range(2):
                pltpu.async_copy(
                    rows_v.at[pl.ds(s * 2 * OCT + q * OCT, OCT)],
                    out_hbm.at[pl.ds(pb, OCT), pl.ds(q * D, D)],
                    osem.at[s],
                )

        def drain_wb(s):
            for q in range(2):
                pltpu.make_async_copy(
                    out_hbm.at[pl.ds(0, OCT), pl.ds(q * D, D)],
                    rows_v.at[pl.ds(s * 2 * OCT + q * OCT, OCT)],
                    osem.at[s],
                ).wait()

        fire_gathers(0, 0)
        fire_gathers(1, 1)

        def step(g, carry):
            s = g % SETS
            s2 = (g + 2) % SETS
            drain_gathers(s)
            fire_wb(g, s)

            @pl.when(g >= 1)
            def _():
                drain_wb(s2)

            @pl.when(g < ngr - 2)
            def _():
                fire_gathers(g + 2, s2)

            return carry

        lax.fori_loop(0, ngr, step, 0)
        drain_wb((ngr - 1) % SETS)

    return gather


_gather_half = _make_gather(NGR_H)


_BH = B1 // NSPLIT


def _unpack_body1(in_ref, out_ref):
    x = in_ref[0]
    out_ref[0, :, 0 : B0 // 2] = x[:, 0:D].T
    out_ref[0, :, B0 // 2 : B0] = x[:, D : 2 * D].T


def _unpack_body2(in_ref, acc_ref, out_ref):
    del acc_ref
    x = in_ref[0]
    out_ref[0, :, 0 : B0 // 2] = x[:, 0:D].T
    out_ref[0, :, B0 // 2 : B0] = x[:, D : 2 * D].T


def _tc_unpack1(pa):
    return pl.pallas_call(
        _unpack_body1,
        grid=(_BH,),
        in_specs=[pl.BlockSpec((1, B0 // 2, 2 * D), lambda b: (b, 0, 0))],
        out_specs=pl.BlockSpec((1, D, B0), lambda b: (b, 0, 0)),
        out_shape=jax.ShapeDtypeStruct((B1, D, B0), jnp.float32),
    )(pa)


def _tc_unpack2(pb, acc):
    return pl.pallas_call(
        _unpack_body2,
        grid=(_BH,),
        in_specs=[
            pl.BlockSpec((1, B0 // 2, 2 * D), lambda b: (b, 0, 0)),
            pl.BlockSpec((1, 8, 128), lambda b: (0, 0, 0)),
        ],
        out_specs=pl.BlockSpec((1, D, B0), lambda b: (b + _BH, 0, 0)),
        out_shape=jax.ShapeDtypeStruct((B1, D, B0), jnp.float32),
        input_output_aliases={1: 0},
    )(pb, acc)


def kernel(x, table):
    xt = x.T
    idxa = _pi(xt[:, : B0 // 2]).reshape(NOCT, OCT)
    idxb = _pi(xt[:, B0 // 2 :]).reshape(NOCT, OCT)

    t_pairs = _tc_pack(table.T)
    tp = t_pairs.reshape(N_EMB, D)
    half = NOCT // NSPLIT
    out_a = _gather_half(idxa[:half], idxb[:half], tp)
    out_b = _gather_half(idxa[half:], idxb[half:], tp)

    pa = out_a.reshape(_BH, B0 // 2, 2 * D)
    pb = out_b.reshape(_BH, B0 // 2, 2 * D)
    res = _tc_unpack2(pb, _tc_unpack1(pa))
    return res.transpose(2, 0, 1)


# speedup vs baseline: 1.0844x; 1.0217x over previous
import functools

import jax
import jax.numpy as jnp
from jax import lax
from jax.experimental import pallas as pl
from jax.experimental.pallas import tpu as pltpu
from jax.experimental.pallas import tpu_sc as plsc

N_EMB = 1000000
D = 64
B0, B1 = 4096, 200
N_ROWS = B0 * B1

_info = plsc.get_sparse_core_info()
NC, NS = _info.num_cores, _info.num_subcores
NW = NC * NS
B_PER_W = N_ROWS // NW
CHUNK = 512
G = 1
GROUP = G * CHUNK
SETS = 3
NG = B_PER_W // GROUP
N_CHUNKS = B_PER_W // CHUNK

_mesh = plsc.VectorSubcoreMesh(core_axis_name="c", subcore_axis_name="s")

_CB = 16384
_NFULL = N_EMB // _CB
_TAIL0 = _NFULL * _CB
_TAILN = N_EMB - _TAIL0
_TH = _TAILN // 2


def _pack_body(in_ref, out_ref):
    c = pl.program_id(0)
    x = in_ref[...]

    @pl.when(c < _NFULL)
    def _():
        out_ref[:, 0:D] = x[:, 0 : _CB // 2].T
        out_ref[:, D : 2 * D] = x[:, _CB // 2 : _CB].T

    @pl.when(c == _NFULL)
    def _():
        out_ref[0:_TH, 0:D] = x[:, 0:_TH].T
        out_ref[0:_TH, D : 2 * D] = x[:, _TH:_TAILN].T


def _tc_pack(table_t):
    return pl.pallas_call(
        _pack_body,
        grid=(_NFULL + 1,),
        in_specs=[pl.BlockSpec((D, _CB), lambda c: (0, c))],
        out_specs=pl.BlockSpec((_CB // 2, 2 * D), lambda c: (c, 0)),
        out_shape=jax.ShapeDtypeStruct((N_EMB // 2, 2 * D), jnp.float32),
    )(table_t)


def _pi(t):
    c, tau = t // _CB, t % _CB
    full = c * _CB + 2 * (tau % (_CB // 2)) + tau // (_CB // 2)
    tau2 = t - _TAIL0
    tail = _TAIL0 + 2 * (tau2 % _TH) + tau2 // _TH
    return jnp.where(t < _TAIL0, full, tail)


NGR = 50
OCT = 256
NOCT = B1 * 8
NSPLIT = 5
NGR_H = NGR // NSPLIT


def _make_gather(ngr):
    @functools.partial(
        pl.kernel,
        mesh=_mesh,
        out_type=jax.ShapeDtypeStruct((ngr * NW * OCT, 2 * D), jnp.float32),
        compiler_params=pltpu.CompilerParams(use_tc_tiling_on_sc=False),
        scratch_types=[
            pltpu.VMEM((ngr, OCT), jnp.int32),
            pltpu.VMEM((ngr, OCT), jnp.int32),
            pltpu.VMEM((SETS * 2 * OCT, D), jnp.float32),
            pltpu.SemaphoreType.DMA((SETS,)),
            pltpu.SemaphoreType.DMA((SETS,)),
        ],
    )
    def gather(idxa_hbm, idxb_hbm, table_hbm, out_hbm, iva, ivb, rows_v,
               gsem, osem):
        wid = lax.axis_index("s") * NC + lax.axis_index("c")
        gbase = wid * ngr
        pltpu.sync_copy(idxa_hbm.at[pl.ds(gbase, ngr)], iva)
        pltpu.sync_copy(idxb_hbm.at[pl.ds(gbase, ngr)], ivb)

        def fire_gathers(g, s):
            pltpu.async_copy(
                table_hbm.at[iva.at[g]],
                rows_v.at[pl.ds(s * 2 * OCT, OCT)],
                gsem.at[s],
            )
            pltpu.async_copy(
                table_hbm.at[ivb.at[g]],
                rows_v.at[pl.ds(s * 2 * OCT + OCT, OCT)],
                gsem.at[s],
            )

        def drain_gathers(s):
            for q in range(2):
                pltpu.make_async_copy(
                    table_hbm.at[pl.ds(0, OCT)],
                    rows_v.at[pl.ds(s * 2 * OCT + q * OCT, OCT)],
                    gsem.at[s],
                ).wait()

        def fire_wb(g, s):
            pb = (gbase + g) * OCT
            for q in range(2):
                pltpu.async_copy(
                    rows_v.at[pl.ds(s * 2 * OCT + q * OCT, OCT)],
                    out_hbm.at[pl.ds(pb, OCT), pl.ds(q * D, D)],
                    osem.at[s],
                )

        def drain_wb(s):
            for q in range(2):
                pltpu.make_async_copy(
                    out_hbm.at[pl.ds(0, OCT), pl.ds(q * D, D)],
                    rows_v.at[pl.ds(s * 2 * OCT + q * OCT, OCT)],
                    osem.at[s],
                ).wait()

        fire_gathers(0, 0)
        fire_gathers(1, 1)

        def step(g, carry):
            s = g % SETS
            s2 = (g + 2) % SETS
            drain_gathers(s)
            fire_wb(g, s)

            @pl.when(g >= 1)
            def _():
                drain_wb(s2)

            @pl.when(g < ngr - 2)
            def _():
                fire_gathers(g + 2, s2)

            return carry

        lax.fori_loop(0, ngr, step, 0)
        drain_wb((ngr - 1) % SETS)

    return gather


_gather_half = _make_gather(NGR_H)


_BH = B1 // NSPLIT


def _unpack_body1(in_ref, out_ref):
    x = in_ref[0]
    out_ref[0, :, 0 : B0 // 2] = x[:, 0:D].T
    out_ref[0, :, B0 // 2 : B0] = x[:, D : 2 * D].T


def _unpack_body2(in_ref, acc_ref, out_ref):
    del acc_ref
    x = in_ref[0]
    out_ref[0, :, 0 : B0 // 2] = x[:, 0:D].T
    out_ref[0, :, B0 // 2 : B0] = x[:, D : 2 * D].T


def _tc_unpack1(pa):
    return pl.pallas_call(
        _unpack_body1,
        grid=(_BH,),
        in_specs=[pl.BlockSpec((1, B0 // 2, 2 * D), lambda b: (b, 0, 0))],
        out_specs=pl.BlockSpec((1, D, B0), lambda b: (b, 0, 0)),
        out_shape=jax.ShapeDtypeStruct((B1, D, B0), jnp.float32),
    )(pa)


def _tc_unpack2(pb, acc, off):
    return pl.pallas_call(
        _unpack_body2,
        grid=(_BH,),
        in_specs=[
            pl.BlockSpec((1, B0 // 2, 2 * D), lambda b: (b, 0, 0)),
            pl.BlockSpec((1, 8, 128), lambda b: (0, 0, 0)),
        ],
        out_specs=pl.BlockSpec((1, D, B0), lambda b, _o=off: (b + _o, 0, 0)),
        out_shape=jax.ShapeDtypeStruct((B1, D, B0), jnp.float32),
        input_output_aliases={1: 0},
    )(pb, acc)


def kernel(x, table):
    xt = x.T
    idxa = _pi(xt[:, : B0 // 2]).reshape(NOCT, OCT)
    idxb = _pi(xt[:, B0 // 2 :]).reshape(NOCT, OCT)

    t_pairs = _tc_pack(table.T)
    tp = t_pairs.reshape(N_EMB, D)
    part = NOCT // NSPLIT
    parts = [
        _gather_half(
            idxa[i * part : (i + 1) * part],
            idxb[i * part : (i + 1) * part],
            tp,
        ).reshape(_BH, B0 // 2, 2 * D)
        for i in range(NSPLIT)
    ]
    res = _tc_unpack1(parts[0])
    for i in range(1, NSPLIT):
        res = _tc_unpack2(parts[i], res, i * _BH)
    return res.transpose(2, 0, 1)


# speedup vs baseline: 1.0989x; 1.0134x over previous
import functools

import jax
import jax.numpy as jnp
from jax import lax
from jax.experimental import pallas as pl
from jax.experimental.pallas import tpu as pltpu
from jax.experimental.pallas import tpu_sc as plsc

N_EMB = 1000000
D = 64
B0, B1 = 4096, 200
N_ROWS = B0 * B1

_info = plsc.get_sparse_core_info()
NC, NS = _info.num_cores, _info.num_subcores
NW = NC * NS
B_PER_W = N_ROWS // NW
CHUNK = 512
G = 1
GROUP = G * CHUNK
SETS = 3
NG = B_PER_W // GROUP
N_CHUNKS = B_PER_W // CHUNK

_mesh = plsc.VectorSubcoreMesh(core_axis_name="c", subcore_axis_name="s")

_CB = 32768
_NFULL = N_EMB // _CB
_TAIL0 = _NFULL * _CB
_TAILN = N_EMB - _TAIL0
_TH = _TAILN // 2


def _pack_body(in_ref, out_ref):
    c = pl.program_id(0)
    x = in_ref[...]

    @pl.when(c < _NFULL)
    def _():
        out_ref[:, 0:D] = x[:, 0 : _CB // 2].T
        out_ref[:, D : 2 * D] = x[:, _CB // 2 : _CB].T

    @pl.when(c == _NFULL)
    def _():
        out_ref[0:_TH, 0:D] = x[:, 0:_TH].T
        out_ref[0:_TH, D : 2 * D] = x[:, _TH:_TAILN].T


def _tc_pack(table_t):
    return pl.pallas_call(
        _pack_body,
        grid=(_NFULL + 1,),
        in_specs=[pl.BlockSpec((D, _CB), lambda c: (0, c))],
        out_specs=pl.BlockSpec((_CB // 2, 2 * D), lambda c: (c, 0)),
        out_shape=jax.ShapeDtypeStruct((N_EMB // 2, 2 * D), jnp.float32),
    )(table_t)


def _pi(t):
    c, tau = t // _CB, t % _CB
    full = c * _CB + 2 * (tau % (_CB // 2)) + tau // (_CB // 2)
    tau2 = t - _TAIL0
    tail = _TAIL0 + 2 * (tau2 % _TH) + tau2 // _TH
    return jnp.where(t < _TAIL0, full, tail)


NGR = 50
OCT = 256
NOCT = B1 * 8
NSPLIT = 5
NGR_H = NGR // NSPLIT


def _make_gather(ngr):
    @functools.partial(
        pl.kernel,
        mesh=_mesh,
        out_type=jax.ShapeDtypeStruct((ngr * NW * OCT, 2 * D), jnp.float32),
        compiler_params=pltpu.CompilerParams(use_tc_tiling_on_sc=False),
        scratch_types=[
            pltpu.VMEM((ngr, OCT), jnp.int32),
            pltpu.VMEM((ngr, OCT), jnp.int32),
            pltpu.VMEM((SETS * 2 * OCT, D), jnp.float32),
            pltpu.SemaphoreType.DMA((SETS,)),
            pltpu.SemaphoreType.DMA((SETS,)),
        ],
    )
    def gather(idxa_hbm, idxb_hbm, table_hbm, out_hbm, iva, ivb, rows_v,
               gsem, osem):
        wid = lax.axis_index("s") * NC + lax.axis_index("c")
        gbase = wid * ngr
        pltpu.sync_copy(idxa_hbm.at[pl.ds(gbase, ngr)], iva)
        pltpu.sync_copy(idxb_hbm.at[pl.ds(gbase, ngr)], ivb)

        def fire_gathers(g, s):
            pltpu.async_copy(
                table_hbm.at[iva.at[g]],
                rows_v.at[pl.ds(s * 2 * OCT, OCT)],
                gsem.at[s],
            )
            pltpu.async_copy(
                table_hbm.at[ivb.at[g]],
                rows_v.at[pl.ds(s * 2 * OCT + OCT, OCT)],
                gsem.at[s],
            )

        def drain_gathers(s):
            for q in range(2):
                pltpu.make_async_copy(
                    table_hbm.at[pl.ds(0, OCT)],
                    rows_v.at[pl.ds(s * 2 * OCT + q * OCT, OCT)],
                    gsem.at[s],
                ).wait()

        def fire_wb(g, s):
            pb = (gbase + g) * OCT
            for q in range(2):
                pltpu.async_copy(
                    rows_v.at[pl.ds(s * 2 * OCT + q * OCT, OCT)],
                    out_hbm.at[pl.ds(pb, OCT), pl.ds(q * D, D)],
                    osem.at[s],
                )

        def drain_wb(s):
            for q in range(2):
                pltpu.make_async_copy(
                    out_hbm.at[pl.ds(0, OCT), pl.ds(q * D, D)],
                    rows_v.at[pl.ds(s * 2 * OCT + q * OCT, OCT)],
                    osem.at[s],
                ).wait()

        fire_gathers(0, 0)
        fire_gathers(1, 1)

        def step(g, carry):
            s = g % SETS
            s2 = (g + 2) % SETS
            drain_gathers(s)
            fire_wb(g, s)

            @pl.when(g >= 1)
            def _():
                drain_wb(s2)

            @pl.when(g < ngr - 2)
            def _():
                fire_gathers(g + 2, s2)

            return carry

        lax.fori_loop(0, ngr, step, 0)
        drain_wb((ngr - 1) % SETS)

    return gather


_gather_half = _make_gather(NGR_H)


_BH = B1 // NSPLIT


def _unpack_body1(in_ref, out_ref):
    x = in_ref[0]
    out_ref[0, :, 0 : B0 // 2] = x[:, 0:D].T
    out_ref[0, :, B0 // 2 : B0] = x[:, D : 2 * D].T


def _unpack_body2(in_ref, acc_ref, out_ref):
    del acc_ref
    x = in_ref[0]
    out_ref[0, :, 0 : B0 // 2] = x[:, 0:D].T
    out_ref[0, :, B0 // 2 : B0] = x[:, D : 2 * D].T


def _tc_unpack1(pa):
    return pl.pallas_call(
        _unpack_body1,
        grid=(_BH,),
        in_specs=[pl.BlockSpec((1, B0 // 2, 2 * D), lambda b: (b, 0, 0))],
        out_specs=pl.BlockSpec((1, D, B0), lambda b: (b, 0, 0)),
        out_shape=jax.ShapeDtypeStruct((B1, D, B0), jnp.float32),
    )(pa)


def _tc_unpack2(pb, acc, off):
    return pl.pallas_call(
        _unpack_body2,
        grid=(_BH,),
        in_specs=[
            pl.BlockSpec((1, B0 // 2, 2 * D), lambda b: (b, 0, 0)),
            pl.BlockSpec((1, 8, 128), lambda b: (0, 0, 0)),
        ],
        out_specs=pl.BlockSpec((1, D, B0), lambda b, _o=off: (b + _o, 0, 0)),
        out_shape=jax.ShapeDtypeStruct((B1, D, B0), jnp.float32),
        input_output_aliases={1: 0},
    )(pb, acc)


def kernel(x, table):
    xt = x.T
    idxa = _pi(xt[:, : B0 // 2]).reshape(NOCT, OCT)
    idxb = _pi(xt[:, B0 // 2 :]).reshape(NOCT, OCT)

    t_pairs = _tc_pack(table.T)
    tp = t_pairs.reshape(N_EMB, D)
    part = NOCT // NSPLIT
    parts = [
        _gather_half(
            idxa[i * part : (i + 1) * part],
            idxb[i * part : (i + 1) * part],
            tp,
        ).reshape(_BH, B0 // 2, 2 * D)
        for i in range(NSPLIT)
    ]
    res = _tc_unpack1(parts[0])
    for i in range(1, NSPLIT):
        res = _tc_unpack2(parts[i], res, i * _BH)
    return res.transpose(2, 0, 1)
